# Initial kernel scaffold; baseline (speedup 1.0000x reference)
import jax
import jax.numpy as jnp
from jax.experimental import pallas as pl


def kernel(input, edge, edge_embed, edge_type, granularity_labels, a, a_2):
    raise NotImplementedError("write your pallas kernel here")


# speedup vs baseline: 66.2251x; 66.2251x over previous
import dataclasses
import functools

import jax
import jax.numpy as jnp
from jax import lax
from jax.experimental import pallas as pl
from jax.experimental.pallas import tpu as pltpu
from jax.experimental.pallas import tpu_sc as plsc

N = 10000
E = 320000
IN = 128
OUT = 128
NREL = 16
ALPHA = 0.2

NC = 2
NS = 16
NW = NC * NS
PE = E // NW
K = 80
NCH = PE // K
NB = 4
G = K // 16
NP = 10240
RPW = NP // NS


def _prep_body(x_ref, a_ref, a2_ref, u_ref, v_ref, uv2_ref):
    x = x_ref[...]
    a1 = a_ref[:, :IN]
    a2w = a_ref[:, IN:]
    dn = (((1,), (1,)), ((), ()))
    u = lax.dot_general(x, a1, dn, preferred_element_type=jnp.float32)
    v = lax.dot_general(x, a2w, dn, preferred_element_type=jnp.float32)
    a2r = a2_ref[...]
    u2 = lax.dot_general(u, a2r, dn, preferred_element_type=jnp.float32)
    v2 = lax.dot_general(v, a2r, dn, preferred_element_type=jnp.float32)
    u_ref[...] = u
    v_ref[...] = v
    pad = jnp.zeros((x.shape[0], OUT - 2), jnp.float32)
    uv2_ref[...] = jnp.concatenate([u2, v2, pad], axis=1)


def _prep(x, a, a_2):
    return pl.pallas_call(
        _prep_body,
        out_shape=(
            jax.ShapeDtypeStruct((N, OUT), jnp.float32),
            jax.ShapeDtypeStruct((N, OUT), jnp.float32),
            jax.ShapeDtypeStruct((N, OUT), jnp.float32),
        ),
    )(x, a, a_2)


def _sc_body(src_h, dst_h, et_h, u2_h, v2_h, glf_h, v_h, zero_h, zero1_h,
             s_out, rs_out, *scr):
    srcc = scr[0:NB]
    dstc = scr[NB:2 * NB]
    etc = scr[2 * NB:3 * NB]
    gis = scr[3 * NB:4 * NB]
    gid = scr[4 * NB:5 * NB]
    gsc = scr[5 * NB:6 * NB]
    gu2 = scr[6 * NB:7 * NB]
    gv2 = scr[7 * NB:8 * NB]
    ggs = scr[8 * NB:9 * NB]
    ggd = scr[9 * NB:10 * NB]
    wv = scr[10 * NB:11 * NB]
    vrow = scr[11 * NB:12 * NB]
    s_sh, rs_sh = scr[12 * NB], scr[12 * NB + 1]
    sem_z = scr[12 * NB + 2]
    sem_ix = scr[12 * NB + 3:12 * NB + 3 + NB]
    sem_g = scr[12 * NB + 3 + NB:12 * NB + 3 + 2 * NB]
    sem_sc = scr[12 * NB + 3 + 2 * NB:12 * NB + 3 + 3 * NB]
    sem_ws = scr[12 * NB + 3 + 3 * NB:12 * NB + 3 + 4 * NB]

    cid = lax.axis_index("c")
    sid = lax.axis_index("s")
    wid = sid * NC + cid
    base = wid * PE

    @pl.when(sid == 0)
    def _():
        pltpu.async_copy(zero_h, s_sh, sem_z).wait()
        pltpu.async_copy(zero1_h, rs_sh, sem_z).wait()

    plsc.subcore_barrier()

    def idx_pair(j, b):
        co = base + j * K
        return (
            (src_h.at[pl.ds(co, K)], srcc[b]),
            (dst_h.at[pl.ds(co, K)], dstc[b]),
            (et_h.at[pl.ds(co, K)], etc[b]),
        )

    def fire_idx(j, b):
        for s, d in idx_pair(j, b):
            pltpu.async_copy(s, d, sem_ix[b])

    def wait_idx(j, b):
        for s, d in idx_pair(j, b):
            pltpu.make_async_copy(s, d, sem_ix[b]).wait()

    def gather_list(b):
        return (
            (u2_h.at[srcc[b]], gu2[b]),
            (v2_h.at[dstc[b]], gv2[b]),
            (glf_h.at[gis[b]], ggs[b]),
            (glf_h.at[gid[b]], ggd[b]),
            (v_h.at[dstc[b]], vrow[b]),
        )

    def fire_gathers(j, b):
        for g in range(G):
            sg = pl.ds(g * 16, 16)
            s16 = srcc[b][sg]
            d16 = dstc[b][sg]
            t16 = etc[b][sg]
            gis[b][sg] = s16 * NREL + t16
            gid[b][sg] = d16 * NREL + t16
            gsc[b][sg] = s16
        for s, d in gather_list(b):
            pltpu.async_copy(s, d, sem_g[b])

    def consume(j, b):
        for s, d in gather_list(b):
            pltpu.make_async_copy(s, d, sem_g[b]).wait()
        for g in range(G):
            sg = pl.ds(g * 16, 16)
            u2s = gu2[b][sg]
            v2d = gv2[b][sg]
            gls = ggs[b][sg]
            gld = ggd[b][sg]
            p = u2s + v2d
            lr = jnp.where(p >= 0.0, p, ALPHA * p)
            w16 = jnp.exp(-lr) * ((gls + gld) * 0.5)
            wv[b][sg] = w16

        @pl.loop(0, K)
        def _(e):
            wsp = plsc.load_gather(wv[b], [jnp.full((16,), e, jnp.int32)])
            for c in range(8):
                slc = pl.ds(c * 16, 16)
                vrow[b][e, slc] = vrow[b][e, slc] * wsp

        pltpu.async_copy(vrow[b], s_sh.at[gsc[b]], sem_sc[b], add=True)
        pltpu.async_copy(wv[b], rs_sh.at[gsc[b]], sem_ws[b], add=True)

    def wait_scatter(b):
        pltpu.make_async_copy(vrow[b], s_sh.at[gsc[b]], sem_sc[b]).wait()
        pltpu.make_async_copy(wv[b], rs_sh.at[gsc[b]], sem_ws[b]).wait()

    fire_idx(0, 0)
    fire_idx(1, 1)
    fire_idx(2, 2)
    wait_idx(0, 0)
    fire_gathers(0, 0)
    wait_idx(1, 1)
    fire_gathers(1, 1)
    fire_idx(3, 3)
    wait_idx(2, 2)
    fire_gathers(2, 2)
    consume(0, 0)
    fire_idx(4, 0)
    wait_idx(3, 3)
    fire_gathers(3, 3)
    consume(1, 1)
    wait_scatter(0)
    fire_idx(5, 1)
    wait_idx(4, 0)
    fire_gathers(4, 0)
    consume(2, 2)
    wait_scatter(1)
    fire_idx(6, 2)
    wait_idx(5, 1)
    fire_gathers(5, 1)
    consume(3, 3)

    def step(j, r):
        rp = (r + 2) % NB
        rn = (r + 3) % NB
        wait_scatter(rp)
        fire_idx(j + 3, rn)
        wait_idx(j + 2, rp)
        fire_gathers(j + 2, rp)
        consume(j, r)

    n_steady = (NCH - 7) // NB

    @pl.loop(0, n_steady)
    def _(i):
        j0 = 4 + NB * i
        for r in range(NB):
            step(j0 + r, r)

    for jj in range(4 + NB * n_steady, NCH - 3):
        step(jj, jj % NB)

    jj = NCH - 3
    rp = (jj + 2) % NB
    wait_scatter(rp)
    wait_idx(jj + 2, rp)
    fire_gathers(jj + 2, rp)
    consume(jj, jj % NB)
    wait_scatter(NCH % NB)
    consume(NCH - 2, (NCH - 2) % NB)
    wait_scatter((NCH + 1) % NB)
    consume(NCH - 1, (NCH - 1) % NB)
    wait_scatter((NCH - 2) % NB)
    wait_scatter((NCH - 1) % NB)

    plsc.subcore_barrier()

    rsl = pl.ds(sid * RPW, RPW)
    pltpu.sync_copy(s_sh.at[rsl], s_out.at[cid, rsl])
    pltpu.sync_copy(rs_sh.at[rsl], rs_out.at[pl.ds(cid * NP + sid * RPW, RPW)])


def _sc_run(src, dst, et, u2, v2, glf, v, zero, zero1):
    mesh = plsc.VectorSubcoreMesh(core_axis_name="c", subcore_axis_name="s",
                                  num_cores=NC, num_subcores=NS)
    scratch = []
    scratch += [pltpu.VMEM((K,), jnp.int32) for _ in range(NB)]
    scratch += [pltpu.VMEM((K,), jnp.int32) for _ in range(NB)]
    scratch += [pltpu.VMEM((K,), jnp.int32) for _ in range(NB)]
    scratch += [pltpu.VMEM((K,), jnp.int32) for _ in range(NB)]
    scratch += [pltpu.VMEM((K,), jnp.int32) for _ in range(NB)]
    scratch += [pltpu.VMEM((K,), jnp.int32) for _ in range(NB)]
    scratch += [pltpu.VMEM((K,), jnp.float32) for _ in range(NB)]
    scratch += [pltpu.VMEM((K,), jnp.float32) for _ in range(NB)]
    scratch += [pltpu.VMEM((K,), jnp.float32) for _ in range(NB)]
    scratch += [pltpu.VMEM((K,), jnp.float32) for _ in range(NB)]
    scratch += [pltpu.VMEM((K,), jnp.float32) for _ in range(NB)]
    scratch += [pltpu.VMEM((K, OUT), jnp.float32) for _ in range(NB)]
    scratch += [
        pltpu.VMEM_SHARED((NP, OUT), jnp.float32),
        pltpu.VMEM_SHARED((NP,), jnp.float32),
        pltpu.SemaphoreType.DMA,
    ]
    scratch += [pltpu.SemaphoreType.DMA for _ in range(4 * NB)]
    cp = pltpu.CompilerParams()
    if "needs_layout_passes" in pltpu.CompilerParams.__dataclass_fields__:
        cp = dataclasses.replace(cp, needs_layout_passes=False)
    if "use_tc_tiling_on_sc" in pltpu.CompilerParams.__dataclass_fields__:
        cp = dataclasses.replace(cp, use_tc_tiling_on_sc=False)
    kern = functools.partial(
        pl.kernel,
        out_type=(
            jax.ShapeDtypeStruct((NC, NP, OUT), jnp.float32),
            jax.ShapeDtypeStruct((NC * NP,), jnp.float32),
        ),
        mesh=mesh,
        scratch_types=scratch,
        compiler_params=cp,
    )(_sc_body)
    return kern(src, dst, et, u2, v2, glf, v, zero, zero1)


def _fin_body(u_ref, s_ref, rs0_ref, rs1_ref, o_ref):
    u = u_ref[...]
    s = s_ref[0, :N, :] + s_ref[1, :N, :]
    rs = rs0_ref[...] + rs1_ref[...]
    h = u * rs + s
    den = jnp.where(rs == 0.0, 1e-12, rs)
    o = h / den
    o_ref[...] = jnp.where(o > 0.0, o, jnp.exp(o) - 1.0)


def _finalize(u, s_parts, rs0, rs1):
    return pl.pallas_call(
        _fin_body,
        out_shape=jax.ShapeDtypeStruct((N, OUT), jnp.float32),
    )(u, s_parts, rs0, rs1)


def kernel(input, edge, edge_embed, edge_type, granularity_labels, a, a_2):
    del edge_embed
    x = input.astype(jnp.float32)
    src = edge[0, :].astype(jnp.int32)
    dst = edge[1, :].astype(jnp.int32)
    et = edge_type.astype(jnp.int32)
    gl = granularity_labels.astype(jnp.float32)

    u, v, uv2 = _prep(x, a.astype(jnp.float32), a_2.astype(jnp.float32))
    u2 = uv2[:, 0]
    v2 = uv2[:, 1]

    glf = gl.reshape(N * NREL)

    zero = jnp.zeros((NP, OUT), jnp.float32)
    zero1 = jnp.zeros((NP,), jnp.float32)
    s_parts, rs_flat = _sc_run(src, dst, et, u2, v2, glf, v, zero, zero1)
    rs0 = rs_flat[:N].reshape(N, 1)
    rs1 = rs_flat[NP:NP + N].reshape(N, 1)
    return _finalize(u, s_parts, rs0, rs1)


# speedup vs baseline: 68.4407x; 1.0335x over previous
import dataclasses
import functools

import jax
import jax.numpy as jnp
from jax import lax
from jax.experimental import pallas as pl
from jax.experimental.pallas import tpu as pltpu
from jax.experimental.pallas import tpu_sc as plsc

N = 10000
E = 320000
IN = 128
OUT = 128
NREL = 16
ALPHA = 0.2

NC = 2
NS = 16
NW = NC * NS
PE = E // NW
K = 80
NCH = PE // K
NB = 4
G = K // 16
NP = 10240
RPW = NP // NS


def _prep_body(x_ref, a_ref, a2_ref, u_ref, v_ref, uv2_ref):
    x = x_ref[...]
    a1 = a_ref[:, :IN]
    a2w = a_ref[:, IN:]
    dn = (((1,), (1,)), ((), ()))
    u = lax.dot_general(x, a1, dn, preferred_element_type=jnp.float32)
    v = lax.dot_general(x, a2w, dn, preferred_element_type=jnp.float32)
    a2r = a2_ref[...]
    u2 = lax.dot_general(u, a2r, dn, preferred_element_type=jnp.float32)
    v2 = lax.dot_general(v, a2r, dn, preferred_element_type=jnp.float32)
    u_ref[...] = u
    v_ref[...] = v
    pad = jnp.zeros((x.shape[0], OUT - 2), jnp.float32)
    uv2_ref[...] = jnp.concatenate([u2, v2, pad], axis=1)


def _prep(x, a, a_2):
    return pl.pallas_call(
        _prep_body,
        out_shape=(
            jax.ShapeDtypeStruct((N, OUT), jnp.float32),
            jax.ShapeDtypeStruct((N, OUT), jnp.float32),
            jax.ShapeDtypeStruct((N, OUT), jnp.float32),
        ),
    )(x, a, a_2)


def _sc_body(src_h, dst_h, et_h, u2_h, v2_h, glf_h, v_h,
             s_out, rs_out, *scr):
    srcc = scr[0:NB]
    dstc = scr[NB:2 * NB]
    etc = scr[2 * NB:3 * NB]
    gis = scr[3 * NB:4 * NB]
    gid = scr[4 * NB:5 * NB]
    gsc = scr[5 * NB:6 * NB]
    gu2 = scr[6 * NB:7 * NB]
    gv2 = scr[7 * NB:8 * NB]
    ggs = scr[8 * NB:9 * NB]
    ggd = scr[9 * NB:10 * NB]
    wv = scr[10 * NB:11 * NB]
    vrow = scr[11 * NB:12 * NB]
    s_sh, rs_sh = scr[12 * NB], scr[12 * NB + 1]
    sem_ix = scr[12 * NB + 2:12 * NB + 2 + NB]
    sem_g = scr[12 * NB + 2 + NB:12 * NB + 2 + 2 * NB]
    sem_sc = scr[12 * NB + 2 + 2 * NB:12 * NB + 2 + 3 * NB]
    sem_ws = scr[12 * NB + 2 + 3 * NB:12 * NB + 2 + 4 * NB]

    cid = lax.axis_index("c")
    sid = lax.axis_index("s")
    wid = sid * NC + cid
    base = wid * PE

    zf = jnp.zeros((16,), jnp.float32)

    @pl.loop(0, K)
    def _(e):
        for c in range(8):
            vrow[0][e, pl.ds(c * 16, 16)] = zf

    for g in range(G):
        wv[0][pl.ds(g * 16, 16)] = zf
    for q in range(RPW // K):
        pltpu.sync_copy(vrow[0], s_sh.at[pl.ds(sid * RPW + q * K, K)])
        pltpu.sync_copy(wv[0], rs_sh.at[pl.ds(sid * RPW + q * K, K)])

    plsc.subcore_barrier()

    def idx_pair(j, b):
        co = base + j * K
        return (
            (src_h.at[pl.ds(co, K)], srcc[b]),
            (dst_h.at[pl.ds(co, K)], dstc[b]),
            (et_h.at[pl.ds(co, K)], etc[b]),
        )

    def fire_idx(j, b):
        for s, d in idx_pair(j, b):
            pltpu.async_copy(s, d, sem_ix[b])

    def wait_idx(j, b):
        for s, d in idx_pair(j, b):
            pltpu.make_async_copy(s, d, sem_ix[b]).wait()

    def gather_list(b):
        return (
            (u2_h.at[srcc[b]], gu2[b]),
            (v2_h.at[dstc[b]], gv2[b]),
            (glf_h.at[gis[b]], ggs[b]),
            (glf_h.at[gid[b]], ggd[b]),
            (v_h.at[dstc[b]], vrow[b]),
        )

    def fire_gathers(j, b):
        for g in range(G):
            sg = pl.ds(g * 16, 16)
            s16 = srcc[b][sg]
            d16 = dstc[b][sg]
            t16 = etc[b][sg]
            gis[b][sg] = s16 * NREL + t16
            gid[b][sg] = d16 * NREL + t16
            gsc[b][sg] = s16
        for s, d in gather_list(b):
            pltpu.async_copy(s, d, sem_g[b])

    def consume(j, b):
        for s, d in gather_list(b):
            pltpu.make_async_copy(s, d, sem_g[b]).wait()
        for g in range(G):
            sg = pl.ds(g * 16, 16)
            u2s = gu2[b][sg]
            v2d = gv2[b][sg]
            gls = ggs[b][sg]
            gld = ggd[b][sg]
            p = u2s + v2d
            lr = jnp.where(p >= 0.0, p, ALPHA * p)
            w16 = jnp.exp(-lr) * ((gls + gld) * 0.5)
            wv[b][sg] = w16

        @pl.loop(0, G)
        def _(g):
            w16 = wv[b][pl.ds(g * 16, 16)]
            for l in range(16):
                wsp = jnp.take_along_axis(
                    w16, jnp.full((16,), l, jnp.int32), axis=0)
                e = g * 16 + l
                for c in range(8):
                    slc = pl.ds(c * 16, 16)
                    vrow[b][e, slc] = vrow[b][e, slc] * wsp

        pltpu.async_copy(vrow[b], s_sh.at[gsc[b]], sem_sc[b], add=True)
        pltpu.async_copy(wv[b], rs_sh.at[gsc[b]], sem_ws[b], add=True)

    def wait_scatter(b):
        pltpu.make_async_copy(vrow[b], s_sh.at[gsc[b]], sem_sc[b]).wait()
        pltpu.make_async_copy(wv[b], rs_sh.at[gsc[b]], sem_ws[b]).wait()

    fire_idx(0, 0)
    fire_idx(1, 1)
    fire_idx(2, 2)
    wait_idx(0, 0)
    fire_gathers(0, 0)
    wait_idx(1, 1)
    fire_gathers(1, 1)
    fire_idx(3, 3)
    wait_idx(2, 2)
    fire_gathers(2, 2)
    consume(0, 0)
    fire_idx(4, 0)
    wait_idx(3, 3)
    fire_gathers(3, 3)
    consume(1, 1)
    wait_scatter(0)
    fire_idx(5, 1)
    wait_idx(4, 0)
    fire_gathers(4, 0)
    consume(2, 2)
    wait_scatter(1)
    fire_idx(6, 2)
    wait_idx(5, 1)
    fire_gathers(5, 1)
    consume(3, 3)

    def step(j, r):
        rp = (r + 2) % NB
        rn = (r + 3) % NB
        wait_scatter(rp)
        fire_idx(j + 3, rn)
        wait_idx(j + 2, rp)
        fire_gathers(j + 2, rp)
        consume(j, r)

    n_steady = (NCH - 7) // NB

    @pl.loop(0, n_steady)
    def _(i):
        j0 = 4 + NB * i
        for r in range(NB):
            step(j0 + r, r)

    for jj in range(4 + NB * n_steady, NCH - 3):
        step(jj, jj % NB)

    jj = NCH - 3
    rp = (jj + 2) % NB
    wait_scatter(rp)
    wait_idx(jj + 2, rp)
    fire_gathers(jj + 2, rp)
    consume(jj, jj % NB)
    wait_scatter(NCH % NB)
    consume(NCH - 2, (NCH - 2) % NB)
    wait_scatter((NCH + 1) % NB)
    consume(NCH - 1, (NCH - 1) % NB)
    wait_scatter((NCH - 2) % NB)
    wait_scatter((NCH - 1) % NB)

    plsc.subcore_barrier()

    rsl = pl.ds(sid * RPW, RPW)
    pltpu.sync_copy(s_sh.at[rsl], s_out.at[cid, rsl])
    pltpu.sync_copy(rs_sh.at[rsl], rs_out.at[pl.ds(cid * NP + sid * RPW, RPW)])


def _sc_run(src, dst, et, u2, v2, glf, v):
    mesh = plsc.VectorSubcoreMesh(core_axis_name="c", subcore_axis_name="s",
                                  num_cores=NC, num_subcores=NS)
    scratch = []
    scratch += [pltpu.VMEM((K,), jnp.int32) for _ in range(NB)]
    scratch += [pltpu.VMEM((K,), jnp.int32) for _ in range(NB)]
    scratch += [pltpu.VMEM((K,), jnp.int32) for _ in range(NB)]
    scratch += [pltpu.VMEM((K,), jnp.int32) for _ in range(NB)]
    scratch += [pltpu.VMEM((K,), jnp.int32) for _ in range(NB)]
    scratch += [pltpu.VMEM((K,), jnp.int32) for _ in range(NB)]
    scratch += [pltpu.VMEM((K,), jnp.float32) for _ in range(NB)]
    scratch += [pltpu.VMEM((K,), jnp.float32) for _ in range(NB)]
    scratch += [pltpu.VMEM((K,), jnp.float32) for _ in range(NB)]
    scratch += [pltpu.VMEM((K,), jnp.float32) for _ in range(NB)]
    scratch += [pltpu.VMEM((K,), jnp.float32) for _ in range(NB)]
    scratch += [pltpu.VMEM((K, OUT), jnp.float32) for _ in range(NB)]
    scratch += [
        pltpu.VMEM_SHARED((NP, OUT), jnp.float32),
        pltpu.VMEM_SHARED((NP,), jnp.float32),
    ]
    scratch += [pltpu.SemaphoreType.DMA for _ in range(4 * NB)]
    cp = pltpu.CompilerParams()
    if "needs_layout_passes" in pltpu.CompilerParams.__dataclass_fields__:
        cp = dataclasses.replace(cp, needs_layout_passes=False)
    if "use_tc_tiling_on_sc" in pltpu.CompilerParams.__dataclass_fields__:
        cp = dataclasses.replace(cp, use_tc_tiling_on_sc=False)
    kern = functools.partial(
        pl.kernel,
        out_type=(
            jax.ShapeDtypeStruct((NC, NP, OUT), jnp.float32),
            jax.ShapeDtypeStruct((NC * NP,), jnp.float32),
        ),
        mesh=mesh,
        scratch_types=scratch,
        compiler_params=cp,
    )(_sc_body)
    return kern(src, dst, et, u2, v2, glf, v)


def _fin_body(u_ref, s_ref, rs0_ref, rs1_ref, o_ref):
    u = u_ref[...]
    s = s_ref[0, :N, :] + s_ref[1, :N, :]
    rs = rs0_ref[...] + rs1_ref[...]
    h = u * rs + s
    den = jnp.where(rs == 0.0, 1e-12, rs)
    o = h / den
    o_ref[...] = jnp.where(o > 0.0, o, jnp.exp(o) - 1.0)


def _finalize(u, s_parts, rs0, rs1):
    return pl.pallas_call(
        _fin_body,
        out_shape=jax.ShapeDtypeStruct((N, OUT), jnp.float32),
    )(u, s_parts, rs0, rs1)


def kernel(input, edge, edge_embed, edge_type, granularity_labels, a, a_2):
    del edge_embed
    x = input.astype(jnp.float32)
    src = edge[0, :].astype(jnp.int32)
    dst = edge[1, :].astype(jnp.int32)
    et = edge_type.astype(jnp.int32)
    gl = granularity_labels.astype(jnp.float32)

    u, v, uv2 = _prep(x, a.astype(jnp.float32), a_2.astype(jnp.float32))
    u2 = uv2[:, 0]
    v2 = uv2[:, 1]

    glf = gl.reshape(N * NREL)

    s_parts, rs_flat = _sc_run(src, dst, et, u2, v2, glf, v)
    rs0 = rs_flat[:N].reshape(N, 1)
    rs1 = rs_flat[NP:NP + N].reshape(N, 1)
    return _finalize(u, s_parts, rs0, rs1)


# speedup vs baseline: 68.4947x; 1.0008x over previous
import dataclasses
import functools

import jax
import jax.numpy as jnp
from jax import lax
from jax.experimental import pallas as pl
from jax.experimental.pallas import tpu as pltpu
from jax.experimental.pallas import tpu_sc as plsc

N = 10000
E = 320000
IN = 128
OUT = 128
NREL = 16
ALPHA = 0.2

NC = 2
NS = 16
NW = NC * NS
PE = E // NW
K = 80
NCH = PE // K
NB = 4
G = K // 16
NP = 10240
RPW = NP // NS


def _prep_body(x_ref, a_ref, a2_ref, u_ref, v_ref, uv2_ref):
    x = x_ref[...]
    a1 = a_ref[:, :IN]
    a2w = a_ref[:, IN:]
    dn = (((1,), (1,)), ((), ()))
    u = lax.dot_general(x, a1, dn, preferred_element_type=jnp.float32)
    v = lax.dot_general(x, a2w, dn, preferred_element_type=jnp.float32)
    a2r = a2_ref[...]
    u2 = lax.dot_general(u, a2r, dn, preferred_element_type=jnp.float32)
    v2 = lax.dot_general(v, a2r, dn, preferred_element_type=jnp.float32)
    u_ref[...] = u
    v_ref[...] = v
    pad = jnp.zeros((x.shape[0], OUT - 2), jnp.float32)
    uv2_ref[...] = jnp.concatenate([u2, v2, pad], axis=1)


def _prep(x, a, a_2):
    return pl.pallas_call(
        _prep_body,
        out_shape=(
            jax.ShapeDtypeStruct((N, OUT), jnp.float32),
            jax.ShapeDtypeStruct((N, OUT), jnp.float32),
            jax.ShapeDtypeStruct((N, OUT), jnp.float32),
        ),
    )(x, a, a_2)


def _sc_body(src_h, dst_h, et_h, u2_h, v2_h, glf_h, v_h,
             s_out, rs_out, *scr):
    srcc = scr[0:NB]
    dstc = scr[NB:2 * NB]
    etc = scr[2 * NB:3 * NB]
    gis = scr[3 * NB:4 * NB]
    gid = scr[4 * NB:5 * NB]
    gsc = scr[5 * NB:6 * NB]
    gu2 = scr[6 * NB:7 * NB]
    gv2 = scr[7 * NB:8 * NB]
    ggs = scr[8 * NB:9 * NB]
    ggd = scr[9 * NB:10 * NB]
    wv = scr[10 * NB:11 * NB]
    vrow = scr[11 * NB:12 * NB]
    s_sh, rs_sh = scr[12 * NB], scr[12 * NB + 1]
    sem_ix = scr[12 * NB + 2:12 * NB + 2 + NB]
    sem_g = scr[12 * NB + 2 + NB:12 * NB + 2 + 2 * NB]
    sem_sc = scr[12 * NB + 2 + 2 * NB:12 * NB + 2 + 3 * NB]
    sem_ws = scr[12 * NB + 2 + 3 * NB:12 * NB + 2 + 4 * NB]

    cid = lax.axis_index("c")
    sid = lax.axis_index("s")
    wid = sid * NC + cid
    base = wid * PE

    zf = jnp.zeros((16,), jnp.float32)

    @pl.loop(0, K)
    def _(e):
        for c in range(8):
            vrow[0][e, pl.ds(c * 16, 16)] = zf

    for g in range(G):
        wv[0][pl.ds(g * 16, 16)] = zf
    for q in range(RPW // K):
        pltpu.sync_copy(vrow[0], s_sh.at[pl.ds(sid * RPW + q * K, K)])
        pltpu.sync_copy(wv[0], rs_sh.at[pl.ds(sid * RPW + q * K, K)])

    plsc.subcore_barrier()

    def idx_pair(j, b):
        co = base + j * K
        return (
            (src_h.at[pl.ds(co, K)], srcc[b]),
            (dst_h.at[pl.ds(co, K)], dstc[b]),
            (et_h.at[pl.ds(co, K)], etc[b]),
        )

    def fire_idx(j, b):
        for s, d in idx_pair(j, b):
            pltpu.async_copy(s, d, sem_ix[b])

    def wait_idx(j, b):
        for s, d in idx_pair(j, b):
            pltpu.make_async_copy(s, d, sem_ix[b]).wait()

    def gather_list(b):
        return (
            (u2_h.at[srcc[b]], gu2[b]),
            (v2_h.at[dstc[b]], gv2[b]),
            (glf_h.at[gis[b]], ggs[b]),
            (glf_h.at[gid[b]], ggd[b]),
            (v_h.at[dstc[b]], vrow[b]),
        )

    def fire_gathers(j, b):
        for g in range(G):
            sg = pl.ds(g * 16, 16)
            s16 = srcc[b][sg]
            d16 = dstc[b][sg]
            t16 = etc[b][sg]
            gis[b][sg] = s16 * NREL + t16
            gid[b][sg] = d16 * NREL + t16
            gsc[b][sg] = s16
        for s, d in gather_list(b):
            pltpu.async_copy(s, d, sem_g[b])

    def consume(j, b):
        for s, d in gather_list(b):
            pltpu.make_async_copy(s, d, sem_g[b]).wait()

        @pl.loop(0, G)
        def _(g):
            sg = pl.ds(g * 16, 16)
            p = gu2[b][sg] + gv2[b][sg]
            lr = jnp.where(p >= 0.0, p, ALPHA * p)
            w16 = jnp.exp(-lr) * ((ggs[b][sg] + ggd[b][sg]) * 0.5)
            wv[b][sg] = w16
            for l in range(16):
                wsp = jnp.take_along_axis(
                    w16, jnp.full((16,), l, jnp.int32), axis=0)
                e = g * 16 + l
                for c in range(8):
                    slc = pl.ds(c * 16, 16)
                    vrow[b][e, slc] = vrow[b][e, slc] * wsp

        pltpu.async_copy(vrow[b], s_sh.at[gsc[b]], sem_sc[b], add=True)
        pltpu.async_copy(wv[b], rs_sh.at[gsc[b]], sem_ws[b], add=True)

    def wait_scatter(b):
        pltpu.make_async_copy(vrow[b], s_sh.at[gsc[b]], sem_sc[b]).wait()
        pltpu.make_async_copy(wv[b], rs_sh.at[gsc[b]], sem_ws[b]).wait()

    fire_idx(0, 0)
    fire_idx(1, 1)
    fire_idx(2, 2)
    wait_idx(0, 0)
    fire_gathers(0, 0)
    wait_idx(1, 1)
    fire_gathers(1, 1)
    fire_idx(3, 3)
    wait_idx(2, 2)
    fire_gathers(2, 2)
    consume(0, 0)
    fire_idx(4, 0)
    wait_idx(3, 3)
    fire_gathers(3, 3)
    consume(1, 1)
    wait_scatter(0)
    fire_idx(5, 1)
    wait_idx(4, 0)
    fire_gathers(4, 0)
    consume(2, 2)
    wait_scatter(1)
    fire_idx(6, 2)
    wait_idx(5, 1)
    fire_gathers(5, 1)
    consume(3, 3)

    def step(j, r):
        rp = (r + 2) % NB
        rn = (r + 3) % NB
        wait_scatter(rp)
        fire_idx(j + 3, rn)
        wait_idx(j + 2, rp)
        fire_gathers(j + 2, rp)
        consume(j, r)

    n_steady = (NCH - 7) // NB

    @pl.loop(0, n_steady)
    def _(i):
        j0 = 4 + NB * i
        for r in range(NB):
            step(j0 + r, r)

    for jj in range(4 + NB * n_steady, NCH - 3):
        step(jj, jj % NB)

    jj = NCH - 3
    rp = (jj + 2) % NB
    wait_scatter(rp)
    wait_idx(jj + 2, rp)
    fire_gathers(jj + 2, rp)
    consume(jj, jj % NB)
    wait_scatter(NCH % NB)
    consume(NCH - 2, (NCH - 2) % NB)
    wait_scatter((NCH + 1) % NB)
    consume(NCH - 1, (NCH - 1) % NB)
    wait_scatter((NCH - 2) % NB)
    wait_scatter((NCH - 1) % NB)

    plsc.subcore_barrier()

    rsl = pl.ds(sid * RPW, RPW)
    pltpu.sync_copy(s_sh.at[rsl], s_out.at[cid, rsl])
    pltpu.sync_copy(rs_sh.at[rsl], rs_out.at[pl.ds(cid * NP + sid * RPW, RPW)])


def _sc_run(src, dst, et, u2, v2, glf, v):
    mesh = plsc.VectorSubcoreMesh(core_axis_name="c", subcore_axis_name="s",
                                  num_cores=NC, num_subcores=NS)
    scratch = []
    scratch += [pltpu.VMEM((K,), jnp.int32) for _ in range(NB)]
    scratch += [pltpu.VMEM((K,), jnp.int32) for _ in range(NB)]
    scratch += [pltpu.VMEM((K,), jnp.int32) for _ in range(NB)]
    scratch += [pltpu.VMEM((K,), jnp.int32) for _ in range(NB)]
    scratch += [pltpu.VMEM((K,), jnp.int32) for _ in range(NB)]
    scratch += [pltpu.VMEM((K,), jnp.int32) for _ in range(NB)]
    scratch += [pltpu.VMEM((K,), jnp.float32) for _ in range(NB)]
    scratch += [pltpu.VMEM((K,), jnp.float32) for _ in range(NB)]
    scratch += [pltpu.VMEM((K,), jnp.float32) for _ in range(NB)]
    scratch += [pltpu.VMEM((K,), jnp.float32) for _ in range(NB)]
    scratch += [pltpu.VMEM((K,), jnp.float32) for _ in range(NB)]
    scratch += [pltpu.VMEM((K, OUT), jnp.float32) for _ in range(NB)]
    scratch += [
        pltpu.VMEM_SHARED((NP, OUT), jnp.float32),
        pltpu.VMEM_SHARED((NP,), jnp.float32),
    ]
    scratch += [pltpu.SemaphoreType.DMA for _ in range(4 * NB)]
    cp = pltpu.CompilerParams()
    if "needs_layout_passes" in pltpu.CompilerParams.__dataclass_fields__:
        cp = dataclasses.replace(cp, needs_layout_passes=False)
    if "use_tc_tiling_on_sc" in pltpu.CompilerParams.__dataclass_fields__:
        cp = dataclasses.replace(cp, use_tc_tiling_on_sc=False)
    kern = functools.partial(
        pl.kernel,
        out_type=(
            jax.ShapeDtypeStruct((NC, NP, OUT), jnp.float32),
            jax.ShapeDtypeStruct((NC * NP,), jnp.float32),
        ),
        mesh=mesh,
        scratch_types=scratch,
        compiler_params=cp,
    )(_sc_body)
    return kern(src, dst, et, u2, v2, glf, v)


def _fin_body(u_ref, s_ref, rs0_ref, rs1_ref, o_ref):
    u = u_ref[...]
    s = s_ref[0] + s_ref[1]
    rs = rs0_ref[...] + rs1_ref[...]
    h = u * rs + s
    den = jnp.where(rs == 0.0, 1e-12, rs)
    o = h / den
    o_ref[...] = jnp.where(o > 0.0, o, jnp.exp(o) - 1.0)


_FB = 1000


def _finalize(u, s_parts, rs0, rs1):
    return pl.pallas_call(
        _fin_body,
        grid=(N // _FB,),
        in_specs=[
            pl.BlockSpec((_FB, OUT), lambda i: (i, 0)),
            pl.BlockSpec((NC, _FB, OUT), lambda i: (0, i, 0)),
            pl.BlockSpec((_FB, 1), lambda i: (i, 0)),
            pl.BlockSpec((_FB, 1), lambda i: (i, 0)),
        ],
        out_specs=pl.BlockSpec((_FB, OUT), lambda i: (i, 0)),
        out_shape=jax.ShapeDtypeStruct((N, OUT), jnp.float32),
    )(u, s_parts, rs0, rs1)


def kernel(input, edge, edge_embed, edge_type, granularity_labels, a, a_2):
    del edge_embed
    x = input.astype(jnp.float32)
    src = edge[0, :].astype(jnp.int32)
    dst = edge[1, :].astype(jnp.int32)
    et = edge_type.astype(jnp.int32)
    gl = granularity_labels.astype(jnp.float32)

    u, v, uv2 = _prep(x, a.astype(jnp.float32), a_2.astype(jnp.float32))
    u2 = uv2[:, 0]
    v2 = uv2[:, 1]

    glf = gl.reshape(N * NREL)

    s_parts, rs_flat = _sc_run(src, dst, et, u2, v2, glf, v)
    rs0 = rs_flat[:N].reshape(N, 1)
    rs1 = rs_flat[NP:NP + N].reshape(N, 1)
    return _finalize(u, s_parts, rs0, rs1)


# speedup vs baseline: 71.0651x; 1.0375x over previous
import dataclasses
import functools

import jax
import jax.numpy as jnp
from jax import lax
from jax.experimental import pallas as pl
from jax.experimental.pallas import tpu as pltpu
from jax.experimental.pallas import tpu_sc as plsc

N = 10000
E = 320000
IN = 128
OUT = 128
NREL = 16
ALPHA = 0.2

NC = 2
NS = 16
NW = NC * NS
PE = E // NW
K = 80
NCH = PE // K
NB = 4
G = K // 16
NP = 10240
RPW = NP // NS


def _prep_body(x_ref, a_ref, a2_ref, u_ref, v_ref, uv2_ref):
    x = x_ref[...]
    a1 = a_ref[:, :IN]
    a2w = a_ref[:, IN:]
    dn = (((1,), (1,)), ((), ()))
    u = lax.dot_general(x, a1, dn, preferred_element_type=jnp.float32)
    v = lax.dot_general(x, a2w, dn, preferred_element_type=jnp.float32)
    a2r = a2_ref[...]
    u2 = lax.dot_general(u, a2r, dn, preferred_element_type=jnp.float32)
    v2 = lax.dot_general(v, a2r, dn, preferred_element_type=jnp.float32)
    u_ref[...] = u
    v_ref[...] = v
    pad = jnp.zeros((x.shape[0], OUT - 2), jnp.float32)
    uv2_ref[...] = jnp.concatenate([u2, v2, pad], axis=1)


def _prep(x, a, a_2):
    return pl.pallas_call(
        _prep_body,
        out_shape=(
            jax.ShapeDtypeStruct((N, OUT), jnp.float32),
            jax.ShapeDtypeStruct((N, OUT), jnp.float32),
            jax.ShapeDtypeStruct((N, OUT), jnp.float32),
        ),
    )(x, a, a_2)


def _sc_body(src_h, dst_h, et_h, ps_h, pd_h, v_h,
             s_out, rs_out, *scr):
    srcc = scr[0:NB]
    dstc = scr[NB:2 * NB]
    etc = scr[2 * NB:3 * NB]
    gis = scr[3 * NB:4 * NB]
    gid = scr[4 * NB:5 * NB]
    gsc = scr[5 * NB:6 * NB]
    gps = scr[6 * NB:7 * NB]
    gpd = scr[7 * NB:8 * NB]
    wv = scr[8 * NB:9 * NB]
    vrow = scr[9 * NB:10 * NB]
    s_sh, rs_sh = scr[10 * NB], scr[10 * NB + 1]
    sem_ix = scr[10 * NB + 2:10 * NB + 2 + NB]
    sem_g = scr[10 * NB + 2 + NB:10 * NB + 2 + 2 * NB]
    sem_sc = scr[10 * NB + 2 + 2 * NB:10 * NB + 2 + 3 * NB]
    sem_ws = scr[10 * NB + 2 + 3 * NB:10 * NB + 2 + 4 * NB]

    cid = lax.axis_index("c")
    sid = lax.axis_index("s")
    wid = sid * NC + cid
    base = wid * PE

    zf = jnp.zeros((16,), jnp.float32)

    @pl.loop(0, K)
    def _(e):
        for c in range(8):
            vrow[0][e, pl.ds(c * 16, 16)] = zf

    for g in range(G):
        wv[0][pl.ds(g * 16, 16)] = zf
    for q in range(RPW // K):
        pltpu.sync_copy(vrow[0], s_sh.at[pl.ds(sid * RPW + q * K, K)])
        pltpu.sync_copy(wv[0], rs_sh.at[pl.ds(sid * RPW + q * K, K)])

    plsc.subcore_barrier()

    def idx_pair(j, b):
        co = base + j * K
        return (
            (src_h.at[pl.ds(co, K)], srcc[b]),
            (dst_h.at[pl.ds(co, K)], dstc[b]),
            (et_h.at[pl.ds(co, K)], etc[b]),
        )

    def fire_idx(j, b):
        for s, d in idx_pair(j, b):
            pltpu.async_copy(s, d, sem_ix[b])

    def wait_idx(j, b):
        for s, d in idx_pair(j, b):
            pltpu.make_async_copy(s, d, sem_ix[b]).wait()

    def gather_list(b):
        return (
            (ps_h.at[gis[b]], gps[b]),
            (pd_h.at[gid[b]], gpd[b]),
            (v_h.at[dstc[b]], vrow[b]),
        )

    def fire_gathers(j, b):
        for g in range(G):
            sg = pl.ds(g * 16, 16)
            s16 = srcc[b][sg]
            d16 = dstc[b][sg]
            t16 = etc[b][sg]
            gis[b][sg] = s16 * NREL + t16
            gid[b][sg] = d16 * NREL + t16
            gsc[b][sg] = s16
        for s, d in gather_list(b):
            pltpu.async_copy(s, d, sem_g[b])

    def consume(j, b):
        for s, d in gather_list(b):
            pltpu.make_async_copy(s, d, sem_g[b]).wait()

        @pl.loop(0, G)
        def _(g):
            sg = pl.ds(g * 16, 16)
            ws_ = gps[b][sg]
            wd_ = gpd[b][sg]
            hi = jnp.int32(-65536)
            u2s = plsc.bitcast(ws_ & hi, jnp.float32)
            v2d = plsc.bitcast(wd_ & hi, jnp.float32)
            gls = plsc.bitcast(ws_ << 16, jnp.float32)
            gld = plsc.bitcast(wd_ << 16, jnp.float32)
            p = u2s + v2d
            lr = jnp.where(p >= 0.0, p, ALPHA * p)
            w16 = jnp.exp(-lr) * ((gls + gld) * 0.5)
            wv[b][sg] = w16
            for l in range(16):
                wsp = jnp.take_along_axis(
                    w16, jnp.full((16,), l, jnp.int32), axis=0)
                e = g * 16 + l
                for c in range(8):
                    slc = pl.ds(c * 16, 16)
                    vrow[b][e, slc] = vrow[b][e, slc] * wsp

        pltpu.async_copy(vrow[b], s_sh.at[gsc[b]], sem_sc[b], add=True)
        pltpu.async_copy(wv[b], rs_sh.at[gsc[b]], sem_ws[b], add=True)

    def wait_scatter(b):
        pltpu.make_async_copy(vrow[b], s_sh.at[gsc[b]], sem_sc[b]).wait()
        pltpu.make_async_copy(wv[b], rs_sh.at[gsc[b]], sem_ws[b]).wait()

    fire_idx(0, 0)
    fire_idx(1, 1)
    fire_idx(2, 2)
    wait_idx(0, 0)
    fire_gathers(0, 0)
    wait_idx(1, 1)
    fire_gathers(1, 1)
    fire_idx(3, 3)
    wait_idx(2, 2)
    fire_gathers(2, 2)
    consume(0, 0)
    fire_idx(4, 0)
    wait_idx(3, 3)
    fire_gathers(3, 3)
    consume(1, 1)
    wait_scatter(0)
    fire_idx(5, 1)
    wait_idx(4, 0)
    fire_gathers(4, 0)
    consume(2, 2)
    wait_scatter(1)
    fire_idx(6, 2)
    wait_idx(5, 1)
    fire_gathers(5, 1)
    consume(3, 3)

    def step(j, r):
        rp = (r + 2) % NB
        rn = (r + 3) % NB
        wait_scatter(rp)
        fire_idx(j + 3, rn)
        wait_idx(j + 2, rp)
        fire_gathers(j + 2, rp)
        consume(j, r)

    n_steady = (NCH - 7) // NB

    @pl.loop(0, n_steady)
    def _(i):
        j0 = 4 + NB * i
        for r in range(NB):
            step(j0 + r, r)

    for jj in range(4 + NB * n_steady, NCH - 3):
        step(jj, jj % NB)

    jj = NCH - 3
    rp = (jj + 2) % NB
    wait_scatter(rp)
    wait_idx(jj + 2, rp)
    fire_gathers(jj + 2, rp)
    consume(jj, jj % NB)
    wait_scatter(NCH % NB)
    consume(NCH - 2, (NCH - 2) % NB)
    wait_scatter((NCH + 1) % NB)
    consume(NCH - 1, (NCH - 1) % NB)
    wait_scatter((NCH - 2) % NB)
    wait_scatter((NCH - 1) % NB)

    plsc.subcore_barrier()

    rsl = pl.ds(sid * RPW, RPW)
    pltpu.sync_copy(s_sh.at[rsl], s_out.at[cid, rsl])
    pltpu.sync_copy(rs_sh.at[rsl], rs_out.at[pl.ds(cid * NP + sid * RPW, RPW)])


def _sc_run(src, dst, et, ps, pd, v):
    mesh = plsc.VectorSubcoreMesh(core_axis_name="c", subcore_axis_name="s",
                                  num_cores=NC, num_subcores=NS)
    scratch = []
    scratch += [pltpu.VMEM((K,), jnp.int32) for _ in range(NB)]
    scratch += [pltpu.VMEM((K,), jnp.int32) for _ in range(NB)]
    scratch += [pltpu.VMEM((K,), jnp.int32) for _ in range(NB)]
    scratch += [pltpu.VMEM((K,), jnp.int32) for _ in range(NB)]
    scratch += [pltpu.VMEM((K,), jnp.int32) for _ in range(NB)]
    scratch += [pltpu.VMEM((K,), jnp.int32) for _ in range(NB)]
    scratch += [pltpu.VMEM((K,), jnp.int32) for _ in range(NB)]
    scratch += [pltpu.VMEM((K,), jnp.int32) for _ in range(NB)]
    scratch += [pltpu.VMEM((K,), jnp.float32) for _ in range(NB)]
    scratch += [pltpu.VMEM((K, OUT), jnp.float32) for _ in range(NB)]
    scratch += [
        pltpu.VMEM_SHARED((NP, OUT), jnp.float32),
        pltpu.VMEM_SHARED((NP,), jnp.float32),
    ]
    scratch += [pltpu.SemaphoreType.DMA for _ in range(4 * NB)]
    cp = pltpu.CompilerParams()
    if "needs_layout_passes" in pltpu.CompilerParams.__dataclass_fields__:
        cp = dataclasses.replace(cp, needs_layout_passes=False)
    if "use_tc_tiling_on_sc" in pltpu.CompilerParams.__dataclass_fields__:
        cp = dataclasses.replace(cp, use_tc_tiling_on_sc=False)
    kern = functools.partial(
        pl.kernel,
        out_type=(
            jax.ShapeDtypeStruct((NC, NP, OUT), jnp.float32),
            jax.ShapeDtypeStruct((NC * NP,), jnp.float32),
        ),
        mesh=mesh,
        scratch_types=scratch,
        compiler_params=cp,
    )(_sc_body)
    return kern(src, dst, et, ps, pd, v)


def _fin_body(u_ref, s_ref, rs0_ref, rs1_ref, o_ref):
    u = u_ref[...]
    s = s_ref[0] + s_ref[1]
    rs = rs0_ref[...] + rs1_ref[...]
    h = u * rs + s
    den = jnp.where(rs == 0.0, 1e-12, rs)
    o = h / den
    o_ref[...] = jnp.where(o > 0.0, o, jnp.exp(o) - 1.0)


_FB = 1000


def _finalize(u, s_parts, rs0, rs1):
    return pl.pallas_call(
        _fin_body,
        grid=(N // _FB,),
        in_specs=[
            pl.BlockSpec((_FB, OUT), lambda i: (i, 0)),
            pl.BlockSpec((NC, _FB, OUT), lambda i: (0, i, 0)),
            pl.BlockSpec((_FB, 1), lambda i: (i, 0)),
            pl.BlockSpec((_FB, 1), lambda i: (i, 0)),
        ],
        out_specs=pl.BlockSpec((_FB, OUT), lambda i: (i, 0)),
        out_shape=jax.ShapeDtypeStruct((N, OUT), jnp.float32),
    )(u, s_parts, rs0, rs1)


def kernel(input, edge, edge_embed, edge_type, granularity_labels, a, a_2):
    del edge_embed
    x = input.astype(jnp.float32)
    src = edge[0, :].astype(jnp.int32)
    dst = edge[1, :].astype(jnp.int32)
    et = edge_type.astype(jnp.int32)
    gl = granularity_labels.astype(jnp.float32)

    u, v, uv2 = _prep(x, a.astype(jnp.float32), a_2.astype(jnp.float32))
    u2 = uv2[:, 0]
    v2 = uv2[:, 1]

    glu = lax.bitcast_convert_type(
        gl.astype(jnp.bfloat16), jnp.uint16).astype(jnp.uint32)
    u2u = lax.bitcast_convert_type(
        u2.astype(jnp.bfloat16), jnp.uint16).astype(jnp.uint32)
    v2u = lax.bitcast_convert_type(
        v2.astype(jnp.bfloat16), jnp.uint16).astype(jnp.uint32)
    ps = lax.bitcast_convert_type(
        (u2u[:, None] << 16) | glu, jnp.int32).reshape(N * NREL)
    pd = lax.bitcast_convert_type(
        (v2u[:, None] << 16) | glu, jnp.int32).reshape(N * NREL)

    s_parts, rs_flat = _sc_run(src, dst, et, ps, pd, v)
    rs0 = rs_flat[:N].reshape(N, 1)
    rs1 = rs_flat[NP:NP + N].reshape(N, 1)
    return _finalize(u, s_parts, rs0, rs1)


# speedup vs baseline: 73.2034x; 1.0301x over previous
import dataclasses
import functools

import jax
import jax.numpy as jnp
from jax import lax
from jax.experimental import pallas as pl
from jax.experimental.pallas import tpu as pltpu
from jax.experimental.pallas import tpu_sc as plsc

N = 10000
E = 320000
IN = 128
OUT = 128
NREL = 16
ALPHA = 0.2

NC = 2
NS = 16
NW = NC * NS
PE = E // NW
K = 80
NCH = PE // K
NB = 4
G = K // 16
NP = 10240
RPW = NP // NS


def _prep_body(x_ref, a_ref, a2_ref, u_ref, v_ref, uv2_ref):
    x = x_ref[...]
    a1 = a_ref[:, :IN]
    a2w = a_ref[:, IN:]
    dn = (((1,), (1,)), ((), ()))
    u = lax.dot_general(x, a1, dn, preferred_element_type=jnp.float32)
    v = lax.dot_general(x, a2w, dn, preferred_element_type=jnp.float32)
    a2r = a2_ref[...]
    u2r = lax.dot_general(a2r, u, dn, preferred_element_type=jnp.float32)
    v2r = lax.dot_general(a2r, v, dn, preferred_element_type=jnp.float32)
    u_ref[...] = u
    v_ref[...] = v
    pad = jnp.zeros((6, x.shape[0]), jnp.float32)
    uv2_ref[...] = jnp.concatenate([u2r, v2r, pad], axis=0)


def _prep(x, a, a_2):
    return pl.pallas_call(
        _prep_body,
        out_shape=(
            jax.ShapeDtypeStruct((N, OUT), jnp.float32),
            jax.ShapeDtypeStruct((N, OUT), jnp.float32),
            jax.ShapeDtypeStruct((8, N), jnp.float32),
        ),
    )(x, a, a_2)


def _sc_body(ew_h, ps_h, pd_h, v_h,
             s_out, rs_out, *scr):
    ewc = scr[0:NB]
    gis = scr[NB:2 * NB]
    gid = scr[2 * NB:3 * NB]
    gsc = scr[3 * NB:4 * NB]
    gdd = scr[4 * NB:5 * NB]
    gps = scr[5 * NB:6 * NB]
    gpd = scr[6 * NB:7 * NB]
    wv = scr[7 * NB:8 * NB]
    vrow = scr[8 * NB:9 * NB]
    s_sh, rs_sh = scr[9 * NB], scr[9 * NB + 1]
    sem_ix = scr[9 * NB + 2:9 * NB + 2 + NB]
    sem_g = scr[9 * NB + 2 + NB:9 * NB + 2 + 2 * NB]
    sem_sc = scr[9 * NB + 2 + 2 * NB:9 * NB + 2 + 3 * NB]
    sem_ws = scr[9 * NB + 2 + 3 * NB:9 * NB + 2 + 4 * NB]

    cid = lax.axis_index("c")
    sid = lax.axis_index("s")
    wid = sid * NC + cid
    base = wid * PE

    zf = jnp.zeros((16,), jnp.float32)

    @pl.loop(0, K)
    def _(e):
        for c in range(8):
            vrow[0][e, pl.ds(c * 16, 16)] = zf

    for g in range(G):
        wv[0][pl.ds(g * 16, 16)] = zf
    for q in range(RPW // K):
        pltpu.sync_copy(vrow[0], s_sh.at[pl.ds(sid * RPW + q * K, K)])
        pltpu.sync_copy(wv[0], rs_sh.at[pl.ds(sid * RPW + q * K, K)])

    plsc.subcore_barrier()

    def fire_idx(j, b):
        co = base + j * K
        pltpu.async_copy(ew_h.at[pl.ds(co, K)], ewc[b], sem_ix[b])

    def wait_idx(j, b):
        co = base + j * K
        pltpu.make_async_copy(ew_h.at[pl.ds(co, K)], ewc[b],
                              sem_ix[b]).wait()

    def gather_list(b):
        return (
            (ps_h.at[gis[b]], gps[b]),
            (pd_h.at[gid[b]], gpd[b]),
            (v_h.at[gdd[b]], vrow[b]),
        )

    def fire_gathers(j, b):
        for g in range(G):
            sg = pl.ds(g * 16, 16)
            w = ewc[b][sg]
            t16 = (w & jnp.uint32(0xF)).astype(jnp.int32)
            d16 = ((w >> 4) & jnp.uint32(0x3FFF)).astype(jnp.int32)
            s16 = (w >> 18).astype(jnp.int32)
            gis[b][sg] = s16 * NREL + t16
            gid[b][sg] = d16 * NREL + t16
            gsc[b][sg] = s16
            gdd[b][sg] = d16
        for s, d in gather_list(b):
            pltpu.async_copy(s, d, sem_g[b])

    def consume(j, b):
        for s, d in gather_list(b):
            pltpu.make_async_copy(s, d, sem_g[b]).wait()

        @pl.loop(0, G)
        def _(g):
            sg = pl.ds(g * 16, 16)
            ws_ = gps[b][sg]
            wd_ = gpd[b][sg]
            hi = jnp.int32(-65536)
            u2s = plsc.bitcast(ws_ & hi, jnp.float32)
            v2d = plsc.bitcast(wd_ & hi, jnp.float32)
            gls = plsc.bitcast(ws_ << 16, jnp.float32)
            gld = plsc.bitcast(wd_ << 16, jnp.float32)
            p = u2s + v2d
            lr = jnp.where(p >= 0.0, p, ALPHA * p)
            w16 = jnp.exp(-lr) * ((gls + gld) * 0.5)
            wv[b][sg] = w16
            for l in range(16):
                wsp = jnp.take_along_axis(
                    w16, jnp.full((16,), l, jnp.int32), axis=0)
                e = g * 16 + l
                for c in range(8):
                    slc = pl.ds(c * 16, 16)
                    vrow[b][e, slc] = vrow[b][e, slc] * wsp

        pltpu.async_copy(vrow[b], s_sh.at[gsc[b]], sem_sc[b], add=True)
        pltpu.async_copy(wv[b], rs_sh.at[gsc[b]], sem_ws[b], add=True)

    def wait_scatter(b):
        pltpu.make_async_copy(vrow[b], s_sh.at[gsc[b]], sem_sc[b]).wait()
        pltpu.make_async_copy(wv[b], rs_sh.at[gsc[b]], sem_ws[b]).wait()

    fire_idx(0, 0)
    fire_idx(1, 1)
    fire_idx(2, 2)
    wait_idx(0, 0)
    fire_gathers(0, 0)
    wait_idx(1, 1)
    fire_gathers(1, 1)
    fire_idx(3, 3)
    wait_idx(2, 2)
    fire_gathers(2, 2)
    consume(0, 0)
    fire_idx(4, 0)
    wait_idx(3, 3)
    fire_gathers(3, 3)
    consume(1, 1)
    wait_scatter(0)
    fire_idx(5, 1)
    wait_idx(4, 0)
    fire_gathers(4, 0)
    consume(2, 2)
    wait_scatter(1)
    fire_idx(6, 2)
    wait_idx(5, 1)
    fire_gathers(5, 1)
    consume(3, 3)

    def step(j, r):
        rp = (r + 2) % NB
        rn = (r + 3) % NB
        wait_scatter(rp)
        fire_idx(j + 3, rn)
        wait_idx(j + 2, rp)
        fire_gathers(j + 2, rp)
        consume(j, r)

    n_steady = (NCH - 7) // NB

    @pl.loop(0, n_steady)
    def _(i):
        j0 = 4 + NB * i
        for r in range(NB):
            step(j0 + r, r)

    for jj in range(4 + NB * n_steady, NCH - 3):
        step(jj, jj % NB)

    jj = NCH - 3
    rp = (jj + 2) % NB
    wait_scatter(rp)
    wait_idx(jj + 2, rp)
    fire_gathers(jj + 2, rp)
    consume(jj, jj % NB)
    wait_scatter(NCH % NB)
    consume(NCH - 2, (NCH - 2) % NB)
    wait_scatter((NCH + 1) % NB)
    consume(NCH - 1, (NCH - 1) % NB)
    wait_scatter((NCH - 2) % NB)
    wait_scatter((NCH - 1) % NB)

    plsc.subcore_barrier()

    rsl = pl.ds(sid * RPW, RPW)
    pltpu.sync_copy(s_sh.at[rsl], s_out.at[cid, rsl])
    pltpu.sync_copy(rs_sh.at[rsl], rs_out.at[pl.ds(cid * NP + sid * RPW, RPW)])


def _sc_run(ew, ps, pd, v):
    mesh = plsc.VectorSubcoreMesh(core_axis_name="c", subcore_axis_name="s",
                                  num_cores=NC, num_subcores=NS)
    scratch = []
    scratch += [pltpu.VMEM((K,), jnp.uint32) for _ in range(NB)]
    scratch += [pltpu.VMEM((K,), jnp.int32) for _ in range(NB)]
    scratch += [pltpu.VMEM((K,), jnp.int32) for _ in range(NB)]
    scratch += [pltpu.VMEM((K,), jnp.int32) for _ in range(NB)]
    scratch += [pltpu.VMEM((K,), jnp.int32) for _ in range(NB)]
    scratch += [pltpu.VMEM((K,), jnp.int32) for _ in range(NB)]
    scratch += [pltpu.VMEM((K,), jnp.int32) for _ in range(NB)]
    scratch += [pltpu.VMEM((K,), jnp.float32) for _ in range(NB)]
    scratch += [pltpu.VMEM((K, OUT), jnp.float32) for _ in range(NB)]
    scratch += [
        pltpu.VMEM_SHARED((NP, OUT), jnp.float32),
        pltpu.VMEM_SHARED((NP,), jnp.float32),
    ]
    scratch += [pltpu.SemaphoreType.DMA for _ in range(4 * NB)]
    cp = pltpu.CompilerParams()
    if "needs_layout_passes" in pltpu.CompilerParams.__dataclass_fields__:
        cp = dataclasses.replace(cp, needs_layout_passes=False)
    if "use_tc_tiling_on_sc" in pltpu.CompilerParams.__dataclass_fields__:
        cp = dataclasses.replace(cp, use_tc_tiling_on_sc=False)
    kern = functools.partial(
        pl.kernel,
        out_type=(
            jax.ShapeDtypeStruct((NC, NP, OUT), jnp.float32),
            jax.ShapeDtypeStruct((NC * NP,), jnp.float32),
        ),
        mesh=mesh,
        scratch_types=scratch,
        compiler_params=cp,
    )(_sc_body)
    return kern(ew, ps, pd, v)


def _fin_body(u_ref, s_ref, rs0_ref, rs1_ref, o_ref):
    u = u_ref[...]
    s = s_ref[0] + s_ref[1]
    rs = rs0_ref[...] + rs1_ref[...]
    h = u * rs + s
    den = jnp.where(rs == 0.0, 1e-12, rs)
    o = h / den
    o_ref[...] = jnp.where(o > 0.0, o, jnp.exp(o) - 1.0)


_FB = 1000


def _finalize(u, s_parts, rs0, rs1):
    return pl.pallas_call(
        _fin_body,
        grid=(N // _FB,),
        in_specs=[
            pl.BlockSpec((_FB, OUT), lambda i: (i, 0)),
            pl.BlockSpec((NC, _FB, OUT), lambda i: (0, i, 0)),
            pl.BlockSpec((_FB, 1), lambda i: (i, 0)),
            pl.BlockSpec((_FB, 1), lambda i: (i, 0)),
        ],
        out_specs=pl.BlockSpec((_FB, OUT), lambda i: (i, 0)),
        out_shape=jax.ShapeDtypeStruct((N, OUT), jnp.float32),
    )(u, s_parts, rs0, rs1)


def kernel(input, edge, edge_embed, edge_type, granularity_labels, a, a_2):
    del edge_embed
    x = input.astype(jnp.float32)
    gl = granularity_labels.astype(jnp.float32)
    ew = ((edge[0, :].astype(jnp.uint32) << 18)
          | (edge[1, :].astype(jnp.uint32) << 4)
          | edge_type.astype(jnp.uint32))

    u, v, uv2t = _prep(x, a.astype(jnp.float32), a_2.astype(jnp.float32))
    u2 = uv2t[0]
    v2 = uv2t[1]

    glu = lax.bitcast_convert_type(
        gl.astype(jnp.bfloat16), jnp.uint16).astype(jnp.uint32)
    u2u = lax.bitcast_convert_type(
        u2.astype(jnp.bfloat16), jnp.uint16).astype(jnp.uint32)
    v2u = lax.bitcast_convert_type(
        v2.astype(jnp.bfloat16), jnp.uint16).astype(jnp.uint32)
    ps = lax.bitcast_convert_type(
        (u2u[:, None] << 16) | glu, jnp.int32).reshape(N * NREL)
    pd = lax.bitcast_convert_type(
        (v2u[:, None] << 16) | glu, jnp.int32).reshape(N * NREL)

    s_parts, rs_flat = _sc_run(ew, ps, pd, v)
    rs0 = rs_flat[:N].reshape(N, 1)
    rs1 = rs_flat[NP:NP + N].reshape(N, 1)
    return _finalize(u, s_parts, rs0, rs1)
